# Initial kernel scaffold; baseline (speedup 1.0000x reference)
import jax
import jax.numpy as jnp
from jax.experimental import pallas as pl


def kernel(query, pos_items, pop_count, num_neg):
    raise NotImplementedError("write your pallas kernel here")


# speedup vs baseline: 511.1329x; 511.1329x over previous
import functools

import jax
import jax.numpy as jnp
import numpy as np
from jax import lax
from jax.experimental import pallas as pl
from jax.experimental.pallas import tpu as pltpu
from jax.experimental.pallas import tpu_sc as plsc

_VOCAB1 = 100001
_PAD = 100016
_NW = 32
_NNEG = 200
_SEARCH_STEPS = 17


def _sampler_call(table_pad, logpp_pad, seeds_flat, pos_items):
    n_seeds = seeds_flat.shape[0]
    n_pos = pos_items.shape[0]
    per_w = n_seeds // _NW
    chunk = 6400
    n_ch = per_w // chunk
    pos_per_w = n_pos // _NW
    mesh = plsc.VectorSubcoreMesh(core_axis_name="c", subcore_axis_name="s")

    @functools.partial(
        pl.kernel,
        mesh=mesh,
        compiler_params=pltpu.CompilerParams(needs_layout_passes=False),
        out_type=(
            jax.ShapeDtypeStruct((n_seeds,), jnp.int32),
            jax.ShapeDtypeStruct((n_seeds,), jnp.float32),
            jax.ShapeDtypeStruct((n_pos,), jnp.float32),
        ),
        scratch_types=[
            pltpu.VMEM((_PAD,), jnp.float32),
            pltpu.VMEM((chunk,), jnp.float32),
            pltpu.VMEM((chunk,), jnp.int32),
            pltpu.VMEM((chunk,), jnp.float32),
            pltpu.VMEM((pos_per_w,), jnp.int32),
            pltpu.VMEM((pos_per_w,), jnp.float32),
        ],
    )
    def body(table_hbm, logpp_hbm, seeds_hbm, pos_hbm,
             negi_hbm, negp_hbm, posp_hbm,
             big_v, seeds_v, idx_v, prob_v, pidx_v, pprob_v):
        wid = lax.axis_index("s") * 2 + lax.axis_index("c")
        base_w = wid * per_w

        pltpu.sync_copy(table_hbm, big_v)
        for c in range(n_ch):
            base = base_w + c * chunk
            pltpu.sync_copy(seeds_hbm.at[pl.ds(base, chunk)], seeds_v)

            def search_step(i, _):
                s = seeds_v[pl.ds(i * 16, 16)]
                lo = jnp.zeros((16,), jnp.int32)
                hi = jnp.full((16,), _VOCAB1, jnp.int32)
                for _u in range(_SEARCH_STEPS):
                    mid = lax.shift_right_logical(lo + hi, 1)
                    t = plsc.load_gather(big_v, [mid])
                    lt = t < s
                    lo = jnp.where(lt, mid + 1, lo)
                    hi = jnp.where(lt, hi, mid)
                idx_v[pl.ds(i * 16, 16)] = lo
                return 0

            lax.fori_loop(0, chunk // 16, search_step, 0)
            pltpu.sync_copy(idx_v, negi_hbm.at[pl.ds(base, chunk)])

        pltpu.sync_copy(logpp_hbm, big_v)
        for c in range(n_ch):
            base = base_w + c * chunk
            pltpu.sync_copy(negi_hbm.at[pl.ds(base, chunk)], idx_v)

            def gather_step(i, _):
                ii = idx_v[pl.ds(i * 16, 16)]
                prob_v[pl.ds(i * 16, 16)] = plsc.load_gather(big_v, [ii])
                return 0

            lax.fori_loop(0, chunk // 16, gather_step, 0)
            pltpu.sync_copy(prob_v, negp_hbm.at[pl.ds(base, chunk)])

        pbase = wid * pos_per_w
        pltpu.sync_copy(pos_hbm.at[pl.ds(pbase, pos_per_w)], pidx_v)

        def pos_step(i, _):
            ii = pidx_v[pl.ds(i * 16, 16)]
            pprob_v[pl.ds(i * 16, 16)] = plsc.load_gather(big_v, [ii])
            return 0

        lax.fori_loop(0, pos_per_w // 16, pos_step, 0)
        pltpu.sync_copy(pprob_v, posp_hbm.at[pl.ds(pbase, pos_per_w)])

    return body(table_pad, logpp_pad, seeds_flat, pos_items)


def kernel(query, pos_items, pop_count, num_neg):
    pc = jnp.log(pop_count + 1.0)
    pc = jnp.concatenate([jnp.ones((1,), dtype=pc.dtype), pc], axis=0)
    pop_prob = pc / jnp.sum(pc)
    table = jnp.cumsum(pop_prob)
    logpp = jnp.log(pop_prob)
    table_pad = jnp.concatenate(
        [table, jnp.full((_PAD - _VOCAB1,), 2.0, jnp.float32)])
    logpp_pad = jnp.concatenate(
        [logpp, jnp.zeros((_PAD - _VOCAB1,), jnp.float32)])

    nq = int(np.prod(query.shape[:-1]))
    seeds = jax.random.uniform(
        jax.random.key(42), (nq, _NNEG), dtype=jnp.float32).reshape(-1)

    negi, negp, posp = _sampler_call(
        table_pad, logpp_pad, seeds, pos_items.astype(jnp.int32))

    neg_items = negi.reshape(tuple(query.shape[:-1]) + (_NNEG,))
    neg_items = neg_items + (num_neg - _NNEG)
    neg_prob = negp.reshape(tuple(query.shape[:-1]) + (_NNEG,))
    return (posp, neg_items, neg_prob)


# speedup vs baseline: 921.7443x; 1.8033x over previous
import functools

import jax
import jax.numpy as jnp
import numpy as np
from jax import lax
from jax.experimental import pallas as pl
from jax.experimental.pallas import tpu as pltpu
from jax.experimental.pallas import tpu_sc as plsc

_VOCAB1 = 100001
_PAD = 100016
_NW = 32
_NNEG = 200
_SEARCH_STEPS = 17
_ILP = 4


def _sampler_call(table_pad, logpp_pad, seeds_flat, pos_items):
    n_seeds = seeds_flat.shape[0]
    n_pos = pos_items.shape[0]
    per_w = n_seeds // _NW
    chunk = 6400
    n_ch = per_w // chunk
    pos_per_w = n_pos // _NW
    mesh = plsc.VectorSubcoreMesh(core_axis_name="c", subcore_axis_name="s")

    @functools.partial(
        pl.kernel,
        mesh=mesh,
        compiler_params=pltpu.CompilerParams(needs_layout_passes=False),
        out_type=(
            jax.ShapeDtypeStruct((n_seeds,), jnp.int32),
            jax.ShapeDtypeStruct((n_seeds,), jnp.float32),
            jax.ShapeDtypeStruct((n_pos,), jnp.float32),
        ),
        scratch_types=[
            pltpu.VMEM((_PAD,), jnp.float32),
            pltpu.VMEM((chunk,), jnp.float32),
            pltpu.VMEM((chunk,), jnp.int32),
            pltpu.VMEM((chunk,), jnp.float32),
            pltpu.VMEM((pos_per_w,), jnp.int32),
            pltpu.VMEM((pos_per_w,), jnp.float32),
        ],
    )
    def body(table_hbm, logpp_hbm, seeds_hbm, pos_hbm,
             negi_hbm, negp_hbm, posp_hbm,
             big_v, seeds_v, idx_v, prob_v, pidx_v, pprob_v):
        wid = lax.axis_index("s") * 2 + lax.axis_index("c")
        base_w = wid * per_w

        pltpu.sync_copy(table_hbm, big_v)
        for c in range(n_ch):
            base = base_w + c * chunk
            pltpu.sync_copy(seeds_hbm.at[pl.ds(base, chunk)], seeds_v)

            def search_step(i, _):
                b0 = i * (16 * _ILP)
                ss = [seeds_v[pl.ds(b0 + u * 16, 16)] for u in range(_ILP)]
                lo = [jnp.zeros((16,), jnp.int32) for _ in range(_ILP)]
                hi = [jnp.full((16,), _VOCAB1, jnp.int32) for _ in range(_ILP)]
                for _u in range(_SEARCH_STEPS):
                    for k in range(_ILP):
                        mid = lax.shift_right_logical(lo[k] + hi[k], 1)
                        t = plsc.load_gather(big_v, [mid])
                        lt = t < ss[k]
                        lo[k] = jnp.where(lt, mid + 1, lo[k])
                        hi[k] = jnp.where(lt, hi[k], mid)
                for u in range(_ILP):
                    idx_v[pl.ds(b0 + u * 16, 16)] = lo[u]
                return 0

            lax.fori_loop(0, chunk // (16 * _ILP), search_step, 0)
            pltpu.sync_copy(idx_v, negi_hbm.at[pl.ds(base, chunk)])

        pltpu.sync_copy(logpp_hbm, big_v)
        for c in range(n_ch):
            base = base_w + c * chunk
            pltpu.sync_copy(negi_hbm.at[pl.ds(base, chunk)], idx_v)

            def gather_step(i, _):
                b0 = i * (16 * _ILP)
                for u in range(_ILP):
                    ii = idx_v[pl.ds(b0 + u * 16, 16)]
                    prob_v[pl.ds(b0 + u * 16, 16)] = plsc.load_gather(
                        big_v, [ii])
                return 0

            lax.fori_loop(0, chunk // (16 * _ILP), gather_step, 0)
            pltpu.sync_copy(prob_v, negp_hbm.at[pl.ds(base, chunk)])

        pbase = wid * pos_per_w
        pltpu.sync_copy(pos_hbm.at[pl.ds(pbase, pos_per_w)], pidx_v)

        def pos_step(i, _):
            ii = pidx_v[pl.ds(i * 16, 16)]
            pprob_v[pl.ds(i * 16, 16)] = plsc.load_gather(big_v, [ii])
            return 0

        lax.fori_loop(0, pos_per_w // 16, pos_step, 0)
        pltpu.sync_copy(pprob_v, posp_hbm.at[pl.ds(pbase, pos_per_w)])

    return body(table_pad, logpp_pad, seeds_flat, pos_items)


def kernel(query, pos_items, pop_count, num_neg):
    pc = jnp.log(pop_count + 1.0)
    pc = jnp.concatenate([jnp.ones((1,), dtype=pc.dtype), pc], axis=0)
    pop_prob = pc / jnp.sum(pc)
    table = jnp.cumsum(pop_prob)
    logpp = jnp.log(pop_prob)
    table_pad = jnp.concatenate(
        [table, jnp.full((_PAD - _VOCAB1,), 2.0, jnp.float32)])
    logpp_pad = jnp.concatenate(
        [logpp, jnp.zeros((_PAD - _VOCAB1,), jnp.float32)])

    nq = int(np.prod(query.shape[:-1]))
    seeds = jax.random.uniform(
        jax.random.key(42), (nq, _NNEG), dtype=jnp.float32).reshape(-1)

    negi, negp, posp = _sampler_call(
        table_pad, logpp_pad, seeds, pos_items.astype(jnp.int32))

    neg_items = negi.reshape(tuple(query.shape[:-1]) + (_NNEG,))
    neg_items = neg_items + (num_neg - _NNEG)
    neg_prob = negp.reshape(tuple(query.shape[:-1]) + (_NNEG,))
    return (posp, neg_items, neg_prob)
